# Initial kernel scaffold; baseline (speedup 1.0000x reference)
import functools

import jax
import jax.numpy as jnp
from jax import lax
from jax.experimental import pallas as pl
from jax.experimental.pallas import tpu as pltpu
from jax.experimental.pallas import tpu_sc as plsc

N = 10000
E = 160000
WIDTH = 32
SW = 48
CDIM = 3

NC = 2
NS = 16
NW = NC * NS
EPAD = 163840
EPW = EPAD // NW
CHUNK = 128
NCHUNK = EPW // CHUNK
KDEPTH = 8
NSTRIPE = N // NS

BE = 1024
GBLKS = EPAD // BE

_MESH = plsc.VectorSubcoreMesh(core_axis_name="c", subcore_axis_name="s")


@functools.partial(
    pl.kernel,
    out_type=jax.ShapeDtypeStruct((EPAD, SW), jnp.float32),
    mesh=_MESH,
    scratch_types=[
        pltpu.VMEM((NCHUNK, CHUNK), jnp.int32),
        pltpu.VMEM((KDEPTH, CHUNK, SW), jnp.float32),
        pltpu.SemaphoreType.DMA,
        pltpu.SemaphoreType.DMA,
    ],
)
def _sc_gather(state_hbm, col2d_hbm, out_hbm, idx_v, buf_v, gsem, ssem):
    w = lax.axis_index("s") * NC + lax.axis_index("c")
    pltpu.sync_copy(col2d_hbm.at[pl.ds(w * NCHUNK, NCHUNK)], idx_v)
    ebase = w * EPW
    for g in range(NCHUNK // KDEPTH):
        gds = []
        for b in range(KDEPTH):
            j = g * KDEPTH + b
            gds.append(
                pltpu.async_copy(state_hbm.at[idx_v.at[j]], buf_v.at[b], gsem))
        for d in gds:
            d.wait()
        sds = []
        for b in range(KDEPTH):
            j = g * KDEPTH + b
            sds.append(
                pltpu.async_copy(
                    buf_v.at[b],
                    out_hbm.at[pl.ds(ebase + j * CHUNK, CHUNK)], ssem))
        for d in sds:
            d.wait()


@functools.partial(
    pl.kernel,
    out_type=jax.ShapeDtypeStruct((NC, N, SW), jnp.float32),
    mesh=_MESH,
    scratch_types=[
        pltpu.VMEM((NCHUNK, CHUNK), jnp.int32),
        pltpu.VMEM((KDEPTH, CHUNK, SW), jnp.float32),
        pltpu.VMEM((NSTRIPE, SW), jnp.float32),
        pltpu.VMEM_SHARED((N, SW), jnp.float32),
        pltpu.SemaphoreType.DMA,
        pltpu.SemaphoreType.DMA,
    ],
)
def _sc_scatter(payload_hbm, row2d_hbm, zeros_hbm, out_hbm,
                idx_v, buf_v, stripe_v, acc_sh, lsem, asem):
    c = lax.axis_index("c")
    s = lax.axis_index("s")
    w = s * NC + c
    pltpu.sync_copy(zeros_hbm, stripe_v)
    pltpu.sync_copy(stripe_v, acc_sh.at[pl.ds(s * NSTRIPE, NSTRIPE)])
    pltpu.sync_copy(row2d_hbm.at[pl.ds(w * NCHUNK, NCHUNK)], idx_v)
    plsc.subcore_barrier()
    ebase = w * EPW
    for g in range(NCHUNK // KDEPTH):
        lds = []
        for b in range(KDEPTH):
            j = g * KDEPTH + b
            lds.append(
                pltpu.async_copy(
                    payload_hbm.at[pl.ds(ebase + j * CHUNK, CHUNK)],
                    buf_v.at[b], lsem))
        for d in lds:
            d.wait()
        ads = []
        for b in range(KDEPTH):
            j = g * KDEPTH + b
            ads.append(
                pltpu.async_copy(buf_v.at[b], acc_sh.at[idx_v.at[j]], asem,
                                 add=True))
        for d in ads:
            d.wait()
    plsc.subcore_barrier()
    pltpu.sync_copy(acc_sh.at[pl.ds(s * NSTRIPE, NSTRIPE)], stripe_v)
    pltpu.sync_copy(stripe_v, out_hbm.at[c, pl.ds(s * NSTRIPE, NSTRIPE)])


def _full(shape):
    return pl.BlockSpec(shape, lambda i: tuple(0 for _ in shape))


def _init_body(x_ref, coords_ref, W1_ref, b1_ref, state_ref):
    h = jnp.dot(x_ref[...], W1_ref[...],
                preferred_element_type=jnp.float32) + b1_ref[...]
    pad = jnp.zeros((N, SW - WIDTH - CDIM), jnp.float32)
    state_ref[...] = jnp.concatenate([h, coords_ref[...], pad], axis=1)


def _payload_body(ea_ref, es_ref, kW1_ref, kb1_ref, kW2_ref, kb2_ref,
                  kW3_ref, kb3_ref, cW1_ref, cb1_ref, cW2_ref, cb2_ref,
                  D_ref, out_ref):
    i = pl.program_id(0)
    k1 = jnp.maximum(
        jnp.dot(ea_ref[...], kW1_ref[...],
                preferred_element_type=jnp.float32) + kb1_ref[...], 0.0)
    k2 = jnp.maximum(
        jnp.dot(k1, kW2_ref[...],
                preferred_element_type=jnp.float32) + kb2_ref[...], 0.0)
    kmat = jnp.dot(k2, kW3_ref[...],
                   preferred_element_type=jnp.float32) + kb3_ref[...]
    es = es_ref[...]
    hcol = es[:, :WIDTH]
    tiled = jnp.concatenate([hcol] * WIDTH, axis=1)
    msg = jnp.dot(kmat * tiled, D_ref[...],
                  preferred_element_type=jnp.float32)
    cf = jnp.maximum(
        jnp.dot(msg, cW1_ref[...],
                preferred_element_type=jnp.float32) + cb1_ref[...], 0.0)
    cfeat = jnp.dot(cf, cW2_ref[...],
                    preferred_element_type=jnp.float32) + cb2_ref[...]
    ccol = es[:, WIDTH:WIDTH + CDIM]
    eidx = i * BE + lax.broadcasted_iota(jnp.int32, (BE, 1), 0)
    valid = (eidx < E).astype(jnp.float32)
    payload = jnp.concatenate(
        [msg, cfeat, ccol * cfeat,
         jnp.ones((BE, 1), jnp.float32),
         jnp.zeros((BE, SW - WIDTH - CDIM - 2), jnp.float32)], axis=1)
    out_ref[...] = payload * valid


def _node_update(state, acc, nW, nb):
    a = acc[0] + acc[1]
    h = state[:, :WIDTH]
    coord = state[:, WIDTH:WIDTH + CDIM]
    inv = 1.0 / jnp.maximum(a[:, 36:37], 1.0)
    h2 = jnp.maximum(
        jnp.dot(h, nW, preferred_element_type=jnp.float32) + nb
        + a[:, :WIDTH] * inv, 0.0)
    coord2 = coord + (coord * a[:, 32:33] - a[:, 33:36]) * inv
    return h2, coord2


def _update_body(state_ref, acc_ref, nW_ref, nb_ref, out_ref):
    h2, coord2 = _node_update(state_ref[...], acc_ref[...], nW_ref[...],
                              nb_ref[...])
    pad = jnp.zeros((N, SW - WIDTH - CDIM), jnp.float32)
    out_ref[...] = jnp.concatenate([h2, coord2, pad], axis=1)


def _final_body(state_ref, acc_ref, nW_ref, nb_ref, fW1_ref, fb1_ref,
                fW2_ref, fb2_ref, out_ref, coord_ref):
    h2, coord2 = _node_update(state_ref[...], acc_ref[...], nW_ref[...],
                              nb_ref[...])
    t = jnp.maximum(
        jnp.dot(h2, fW1_ref[...],
                preferred_element_type=jnp.float32) + fb1_ref[...], 0.0)
    out_ref[...] = jnp.dot(t, fW2_ref[...],
                           preferred_element_type=jnp.float32) + fb2_ref[...]
    coord_ref[...] = coord2


def _tc_init(x, coords, W1, b1):
    return pl.pallas_call(
        _init_body,
        out_shape=jax.ShapeDtypeStruct((N, SW), jnp.float32),
        grid=(1,),
        in_specs=[_full((N, 6)), _full((N, CDIM)), _full((6, WIDTH)),
                  _full((1, WIDTH))],
        out_specs=_full((N, SW)),
    )(x, coords, W1, b1)


def _tc_payload(ea, es, kW1, kb1, kW2, kb2, kW3, kb3, cW1, cb1, cW2, cb2, D):
    return pl.pallas_call(
        _payload_body,
        out_shape=jax.ShapeDtypeStruct((EPAD, SW), jnp.float32),
        grid=(GBLKS,),
        in_specs=[
            pl.BlockSpec((BE, 4), lambda i: (i, 0)),
            pl.BlockSpec((BE, SW), lambda i: (i, 0)),
            _full((4, 64)), _full((1, 64)),
            _full((64, 128)), _full((1, 128)),
            _full((128, 1024)), _full((1, 1024)),
            _full((WIDTH, WIDTH)), _full((1, WIDTH)),
            _full((WIDTH, 1)), _full((1, 1)),
            _full((1024, WIDTH)),
        ],
        out_specs=pl.BlockSpec((BE, SW), lambda i: (i, 0)),
    )(ea, es, kW1, kb1, kW2, kb2, kW3, kb3, cW1, cb1, cW2, cb2, D)


def _tc_update(state, acc, nW, nb):
    return pl.pallas_call(
        _update_body,
        out_shape=jax.ShapeDtypeStruct((N, SW), jnp.float32),
        grid=(1,),
        in_specs=[_full((N, SW)), _full((NC, N, SW)),
                  _full((WIDTH, WIDTH)), _full((1, WIDTH))],
        out_specs=_full((N, SW)),
    )(state, acc, nW, nb)


def _tc_final(state, acc, nW, nb, fW1, fb1, fW2, fb2):
    return pl.pallas_call(
        _final_body,
        out_shape=(jax.ShapeDtypeStruct((N, 1), jnp.float32),
                   jax.ShapeDtypeStruct((N, CDIM), jnp.float32)),
        grid=(1,),
        in_specs=[_full((N, SW)), _full((NC, N, SW)),
                  _full((WIDTH, WIDTH)), _full((1, WIDTH)),
                  _full((WIDTH, 2 * WIDTH)), _full((1, 2 * WIDTH)),
                  _full((2 * WIDTH, 1)), _full((1, 1))],
        out_specs=(_full((N, 1)), _full((N, CDIM))),
    )(state, acc, nW, nb, fW1, fb1, fW2, fb2)


@jax.jit
def kernel(x, edge_index, edge_attr, coords_init,
           W1, b1, kW1, kb1, kW2, kb2, kW3, kb3,
           nW, nb, cW1, cb1, cW2, cb2, fW1, fb1, fW2, fb2):
    row = edge_index[0]
    col = edge_index[1]
    col2d = jnp.pad(col, (0, EPAD - E)).reshape(EPAD // CHUNK, CHUNK)
    row2d = jnp.pad(row, (0, EPAD - E)).reshape(EPAD // CHUNK, CHUNK)
    ea = jnp.pad(edge_attr, ((0, EPAD - E), (0, 0)))
    zeros_stripe = jnp.zeros((NSTRIPE, SW), jnp.float32)
    D = jnp.repeat(jnp.eye(WIDTH, dtype=jnp.float32), WIDTH, axis=0)
    b1r = b1.reshape(1, -1)
    kb1r = kb1.reshape(1, -1)
    kb2r = kb2.reshape(1, -1)
    kb3r = kb3.reshape(1, -1)
    cb1r = cb1.reshape(1, -1)
    cb2r = cb2.reshape(1, -1)
    nbr = nb.reshape(1, -1)
    fb1r = fb1.reshape(1, -1)
    fb2r = fb2.reshape(1, -1)

    state = _tc_init(x, coords_init, W1, b1r)
    for d in range(2):
        es = _sc_gather(state, col2d)
        payload = _tc_payload(ea, es, kW1, kb1r, kW2, kb2r, kW3, kb3r,
                              cW1, cb1r, cW2, cb2r, D)
        acc = _sc_scatter(payload, row2d, zeros_stripe)
        if d == 0:
            state = _tc_update(state, acc, nW, nbr)
        else:
            out, coord = _tc_final(state, acc, nW, nbr, fW1, fb1r,
                                   fW2, fb2r)
    return out, coord


# speedup vs baseline: 3.0726x; 3.0726x over previous
import functools

import jax
import jax.numpy as jnp
from jax import lax
from jax.experimental import pallas as pl
from jax.experimental.pallas import tpu as pltpu
from jax.experimental.pallas import tpu_sc as plsc

N = 10000
E = 160000
WIDTH = 32
SW = 48
STW = 128
ESW = 64
CDIM = 3

NC = 2
NS = 16
NW = NC * NS
EPAD = 163840
EPW = EPAD // NW
CHUNK = 128
NCHUNK = EPW // CHUNK
KDEPTH = 8
GDEPTH = 4
NP = 10240
NSTRIPE = NP // NS

BE = 1024
GBLKS = EPAD // BE

_MESH = plsc.VectorSubcoreMesh(core_axis_name="c", subcore_axis_name="s")


@functools.partial(
    pl.kernel,
    out_type=jax.ShapeDtypeStruct((EPAD, STW), jnp.float32),
    mesh=_MESH,
    scratch_types=[
        pltpu.VMEM((NCHUNK, CHUNK), jnp.int32),
        pltpu.VMEM((GDEPTH, CHUNK, STW), jnp.float32),
        pltpu.SemaphoreType.DMA,
        pltpu.SemaphoreType.DMA,
    ],
)
def _sc_gather(state_hbm, col2d_hbm, out_hbm, idx_v, buf_v, gsem, ssem):
    w = lax.axis_index("s") * NC + lax.axis_index("c")
    pltpu.sync_copy(col2d_hbm.at[pl.ds(w * NCHUNK, NCHUNK)], idx_v)
    ebase = w * EPW
    for g in range(NCHUNK // GDEPTH):
        gds = []
        for b in range(GDEPTH):
            j = g * GDEPTH + b
            gds.append(
                pltpu.async_copy(state_hbm.at[idx_v.at[j]], buf_v.at[b], gsem))
        for d in gds:
            d.wait()
        sds = []
        for b in range(GDEPTH):
            j = g * GDEPTH + b
            sds.append(
                pltpu.async_copy(
                    buf_v.at[b],
                    out_hbm.at[pl.ds(ebase + j * CHUNK, CHUNK)], ssem))
        for d in sds:
            d.wait()


@functools.partial(
    pl.kernel,
    out_type=jax.ShapeDtypeStruct((NC, NP, SW), jnp.float32),
    mesh=_MESH,
    compiler_params=pltpu.CompilerParams(use_tc_tiling_on_sc=False),
    scratch_types=[
        pltpu.VMEM((NCHUNK, CHUNK), jnp.int32),
        pltpu.VMEM((KDEPTH, CHUNK, SW), jnp.float32),
        pltpu.VMEM((NSTRIPE, SW), jnp.float32),
        pltpu.VMEM_SHARED((NP, SW), jnp.float32),
        pltpu.SemaphoreType.DMA,
        pltpu.SemaphoreType.DMA,
    ],
)
def _sc_scatter(payload_hbm, row2d_hbm, zeros_hbm, out_hbm,
                idx_v, buf_v, stripe_v, acc_sh, lsem, asem):
    c = lax.axis_index("c")
    s = lax.axis_index("s")
    w = s * NC + c
    pltpu.sync_copy(zeros_hbm, stripe_v)
    pltpu.sync_copy(stripe_v, acc_sh.at[pl.ds(s * NSTRIPE, NSTRIPE)])
    pltpu.sync_copy(row2d_hbm.at[pl.ds(w * NCHUNK, NCHUNK)], idx_v)
    plsc.subcore_barrier()
    ebase = w * EPW
    for g in range(NCHUNK // KDEPTH):
        lds = []
        for b in range(KDEPTH):
            j = g * KDEPTH + b
            lds.append(
                pltpu.async_copy(
                    payload_hbm.at[pl.ds(ebase + j * CHUNK, CHUNK)],
                    buf_v.at[b], lsem))
        for d in lds:
            d.wait()
        ads = []
        for b in range(KDEPTH):
            j = g * KDEPTH + b
            ads.append(
                pltpu.async_copy(buf_v.at[b], acc_sh.at[idx_v.at[j]], asem,
                                 add=True))
        for d in ads:
            d.wait()
    plsc.subcore_barrier()
    pltpu.sync_copy(acc_sh.at[pl.ds(s * NSTRIPE, NSTRIPE)], stripe_v)
    pltpu.sync_copy(stripe_v, out_hbm.at[c, pl.ds(s * NSTRIPE, NSTRIPE)])


def _full(shape):
    return pl.BlockSpec(shape, lambda i: tuple(0 for _ in shape))


def _init_body(x_ref, coords_ref, W1_ref, b1_ref, state_ref):
    h = jnp.dot(x_ref[...], W1_ref[...],
                preferred_element_type=jnp.float32) + b1_ref[...]
    pad = jnp.zeros((N, STW - WIDTH - CDIM), jnp.float32)
    state_ref[...] = jnp.concatenate([h, coords_ref[...], pad], axis=1)


def _payload_body(ea_ref, es_ref, kW1_ref, kb1_ref, kW2_ref, kb2_ref,
                  kW3_ref, kb3_ref, cW1_ref, cb1_ref, cW2_ref, cb2_ref,
                  D_ref, out_ref):
    i = pl.program_id(0)
    k1 = jnp.maximum(
        jnp.dot(ea_ref[...], kW1_ref[...],
                preferred_element_type=jnp.float32) + kb1_ref[...], 0.0)
    k2 = jnp.maximum(
        jnp.dot(k1, kW2_ref[...],
                preferred_element_type=jnp.float32) + kb2_ref[...], 0.0)
    kmat = jnp.dot(k2, kW3_ref[...],
                   preferred_element_type=jnp.float32) + kb3_ref[...]
    es = es_ref[...]
    hcol = es[:, :WIDTH]
    tiled = jnp.concatenate([hcol] * WIDTH, axis=1)
    msg = jnp.dot(kmat * tiled, D_ref[...],
                  preferred_element_type=jnp.float32)
    cf = jnp.maximum(
        jnp.dot(msg, cW1_ref[...],
                preferred_element_type=jnp.float32) + cb1_ref[...], 0.0)
    cfeat = jnp.dot(cf, cW2_ref[...],
                    preferred_element_type=jnp.float32) + cb2_ref[...]
    ccol = es[:, WIDTH:WIDTH + CDIM]
    eidx = i * BE + lax.broadcasted_iota(jnp.int32, (BE, 1), 0)
    valid = (eidx < E).astype(jnp.float32)
    payload = jnp.concatenate(
        [msg, cfeat, ccol * cfeat,
         jnp.ones((BE, 1), jnp.float32),
         jnp.zeros((BE, SW - WIDTH - CDIM - 2), jnp.float32)], axis=1)
    out_ref[...] = payload * valid


def _node_update(state, acc, nW, nb):
    a = (acc[0] + acc[1])[:N]
    h = state[:, :WIDTH]
    coord = state[:, WIDTH:WIDTH + CDIM]
    inv = 1.0 / jnp.maximum(a[:, 36:37], 1.0)
    h2 = jnp.maximum(
        jnp.dot(h, nW, preferred_element_type=jnp.float32) + nb
        + a[:, :WIDTH] * inv, 0.0)
    coord2 = coord + (coord * a[:, 32:33] - a[:, 33:36]) * inv
    return h2, coord2


def _update_body(state_ref, acc_ref, nW_ref, nb_ref, out_ref):
    h2, coord2 = _node_update(state_ref[...], acc_ref[...], nW_ref[...],
                              nb_ref[...])
    pad = jnp.zeros((N, STW - WIDTH - CDIM), jnp.float32)
    out_ref[...] = jnp.concatenate([h2, coord2, pad], axis=1)


def _final_body(state_ref, acc_ref, nW_ref, nb_ref, fW1_ref, fb1_ref,
                fW2_ref, fb2_ref, out_ref, coord_ref):
    h2, coord2 = _node_update(state_ref[...], acc_ref[...], nW_ref[...],
                              nb_ref[...])
    t = jnp.maximum(
        jnp.dot(h2, fW1_ref[...],
                preferred_element_type=jnp.float32) + fb1_ref[...], 0.0)
    out_ref[...] = jnp.dot(t, fW2_ref[...],
                           preferred_element_type=jnp.float32) + fb2_ref[...]
    coord_ref[...] = coord2


def _tc_init(x, coords, W1, b1):
    return pl.pallas_call(
        _init_body,
        out_shape=jax.ShapeDtypeStruct((N, STW), jnp.float32),
        grid=(1,),
        in_specs=[_full((N, 6)), _full((N, CDIM)), _full((6, WIDTH)),
                  _full((1, WIDTH))],
        out_specs=_full((N, STW)),
    )(x, coords, W1, b1)


def _tc_payload(ea, es, kW1, kb1, kW2, kb2, kW3, kb3, cW1, cb1, cW2, cb2, D):
    return pl.pallas_call(
        _payload_body,
        out_shape=jax.ShapeDtypeStruct((EPAD, SW), jnp.float32),
        grid=(GBLKS,),
        in_specs=[
            pl.BlockSpec((BE, 4), lambda i: (i, 0)),
            pl.BlockSpec((BE, STW), lambda i: (i, 0)),
            _full((4, 64)), _full((1, 64)),
            _full((64, 128)), _full((1, 128)),
            _full((128, 1024)), _full((1, 1024)),
            _full((WIDTH, WIDTH)), _full((1, WIDTH)),
            _full((WIDTH, 1)), _full((1, 1)),
            _full((1024, WIDTH)),
        ],
        out_specs=pl.BlockSpec((BE, SW), lambda i: (i, 0)),
    )(ea, es, kW1, kb1, kW2, kb2, kW3, kb3, cW1, cb1, cW2, cb2, D)


def _tc_update(state, acc, nW, nb):
    return pl.pallas_call(
        _update_body,
        out_shape=jax.ShapeDtypeStruct((N, STW), jnp.float32),
        grid=(1,),
        in_specs=[_full((N, STW)), _full((NC, NP, SW)),
                  _full((WIDTH, WIDTH)), _full((1, WIDTH))],
        out_specs=_full((N, STW)),
    )(state, acc, nW, nb)


def _tc_final(state, acc, nW, nb, fW1, fb1, fW2, fb2):
    return pl.pallas_call(
        _final_body,
        out_shape=(jax.ShapeDtypeStruct((N, 1), jnp.float32),
                   jax.ShapeDtypeStruct((N, CDIM), jnp.float32)),
        grid=(1,),
        in_specs=[_full((N, STW)), _full((NC, NP, SW)),
                  _full((WIDTH, WIDTH)), _full((1, WIDTH)),
                  _full((WIDTH, 2 * WIDTH)), _full((1, 2 * WIDTH)),
                  _full((2 * WIDTH, 1)), _full((1, 1))],
        out_specs=(_full((N, 1)), _full((N, CDIM))),
    )(state, acc, nW, nb, fW1, fb1, fW2, fb2)


@jax.jit
def kernel(x, edge_index, edge_attr, coords_init,
           W1, b1, kW1, kb1, kW2, kb2, kW3, kb3,
           nW, nb, cW1, cb1, cW2, cb2, fW1, fb1, fW2, fb2):
    row = edge_index[0]
    col = edge_index[1]
    col2d = jnp.pad(col, (0, EPAD - E)).reshape(EPAD // CHUNK, CHUNK)
    row2d = jnp.pad(row, (0, EPAD - E)).reshape(EPAD // CHUNK, CHUNK)
    ea = jnp.pad(edge_attr, ((0, EPAD - E), (0, 0)))
    zeros_stripe = jnp.zeros((NSTRIPE, SW), jnp.float32)
    D = jnp.repeat(jnp.eye(WIDTH, dtype=jnp.float32), WIDTH, axis=0)
    b1r = b1.reshape(1, -1)
    kb1r = kb1.reshape(1, -1)
    kb2r = kb2.reshape(1, -1)
    kb3r = kb3.reshape(1, -1)
    cb1r = cb1.reshape(1, -1)
    cb2r = cb2.reshape(1, -1)
    nbr = nb.reshape(1, -1)
    fb1r = fb1.reshape(1, -1)
    fb2r = fb2.reshape(1, -1)

    state = _tc_init(x, coords_init, W1, b1r)
    for d in range(2):
        es = _sc_gather(state, col2d)
        payload = _tc_payload(ea, es, kW1, kb1r, kW2, kb2r, kW3, kb3r,
                              cW1, cb1r, cW2, cb2r, D)
        acc = _sc_scatter(payload, row2d, zeros_stripe)
        if d == 0:
            state = _tc_update(state, acc, nW, nbr)
        else:
            out, coord = _tc_final(state, acc, nW, nbr, fW1, fb1r,
                                   fW2, fb2r)
    return out, coord


# speedup vs baseline: 3.4274x; 1.1155x over previous
import functools

import jax
import jax.numpy as jnp
from jax import lax
from jax.experimental import pallas as pl
from jax.experimental.pallas import tpu as pltpu
from jax.experimental.pallas import tpu_sc as plsc

N = 10000
E = 160000
WIDTH = 32
SW = 48
CDIM = 3

NC = 2
NS = 16
NW = NC * NS
EPAD = 163840
EPW = EPAD // NW
CHUNK = 128
NCHUNK = EPW // CHUNK
KDEPTH = 8
NP = 10240
NSTRIPE = NP // NS

BE = 1024
GBLKS = EPAD // BE

_MESH = plsc.VectorSubcoreMesh(core_axis_name="c", subcore_axis_name="s")


@functools.partial(
    pl.kernel,
    out_type=jax.ShapeDtypeStruct((EPAD, SW), jnp.float32),
    mesh=_MESH,
    compiler_params=pltpu.CompilerParams(use_tc_tiling_on_sc=False),
    scratch_types=[
        pltpu.VMEM((NCHUNK, CHUNK), jnp.int32),
        pltpu.VMEM((KDEPTH, CHUNK, SW), jnp.float32),
        pltpu.SemaphoreType.DMA,
        pltpu.SemaphoreType.DMA,
    ],
)
def _sc_gather(state_hbm, col2d_hbm, out_hbm, idx_v, buf_v, gsem, ssem):
    w = lax.axis_index("s") * NC + lax.axis_index("c")
    pltpu.sync_copy(col2d_hbm.at[pl.ds(w * NCHUNK, NCHUNK)], idx_v)
    ebase = w * EPW
    LAG = KDEPTH - 1
    gds = [None] * NCHUNK
    sds = [None] * NCHUNK

    def _store(i):
        return pltpu.async_copy(
            buf_v.at[i % KDEPTH],
            out_hbm.at[pl.ds(ebase + i * CHUNK, CHUNK)], ssem)

    for j in range(NCHUNK):
        if j >= KDEPTH:
            sds[j - KDEPTH].wait()
        gds[j] = pltpu.async_copy(state_hbm.at[idx_v.at[j]],
                                  buf_v.at[j % KDEPTH], gsem)
        i = j - LAG
        if i >= 0:
            gds[i].wait()
            sds[i] = _store(i)
    for i in range(NCHUNK - LAG, NCHUNK):
        gds[i].wait()
        sds[i] = _store(i)
    for i in range(NCHUNK - KDEPTH, NCHUNK):
        sds[i].wait()


@functools.partial(
    pl.kernel,
    out_type=jax.ShapeDtypeStruct((NC, NP, SW), jnp.float32),
    mesh=_MESH,
    compiler_params=pltpu.CompilerParams(use_tc_tiling_on_sc=False),
    scratch_types=[
        pltpu.VMEM((NCHUNK, CHUNK), jnp.int32),
        pltpu.VMEM((KDEPTH, CHUNK, SW), jnp.float32),
        pltpu.VMEM((NSTRIPE, SW), jnp.float32),
        pltpu.VMEM_SHARED((NP, SW), jnp.float32),
        pltpu.SemaphoreType.DMA,
        pltpu.SemaphoreType.DMA,
    ],
)
def _sc_scatter(payload_hbm, row2d_hbm, zeros_hbm, out_hbm,
                idx_v, buf_v, stripe_v, acc_sh, lsem, asem):
    c = lax.axis_index("c")
    s = lax.axis_index("s")
    w = s * NC + c
    pltpu.sync_copy(zeros_hbm, stripe_v)
    pltpu.sync_copy(stripe_v, acc_sh.at[pl.ds(s * NSTRIPE, NSTRIPE)])
    pltpu.sync_copy(row2d_hbm.at[pl.ds(w * NCHUNK, NCHUNK)], idx_v)
    plsc.subcore_barrier()
    ebase = w * EPW
    for g in range(NCHUNK // KDEPTH):
        lds = []
        for b in range(KDEPTH):
            j = g * KDEPTH + b
            lds.append(
                pltpu.async_copy(
                    payload_hbm.at[pl.ds(ebase + j * CHUNK, CHUNK)],
                    buf_v.at[b], lsem))
        for d in lds:
            d.wait()
        ads = []
        for b in range(KDEPTH):
            j = g * KDEPTH + b
            ads.append(
                pltpu.async_copy(buf_v.at[b], acc_sh.at[idx_v.at[j]], asem,
                                 add=True))
        for d in ads:
            d.wait()
    plsc.subcore_barrier()
    pltpu.sync_copy(acc_sh.at[pl.ds(s * NSTRIPE, NSTRIPE)], stripe_v)
    pltpu.sync_copy(stripe_v, out_hbm.at[c, pl.ds(s * NSTRIPE, NSTRIPE)])


def _full(shape):
    return pl.BlockSpec(shape, lambda i: tuple(0 for _ in shape))


def _init_body(x_ref, coords_ref, W1_ref, b1_ref, state_ref):
    h = jnp.dot(x_ref[...], W1_ref[...],
                preferred_element_type=jnp.float32) + b1_ref[...]
    pad = jnp.zeros((N, SW - WIDTH - CDIM), jnp.float32)
    state_ref[...] = jnp.concatenate([h, coords_ref[...], pad], axis=1)


def _payload_body(ea_ref, es_ref, kW1_ref, kb1_ref, kW2_ref, kb2_ref,
                  kW3_ref, kb3_ref, cW1_ref, cb1_ref, cW2_ref, cb2_ref,
                  D_ref, out_ref):
    i = pl.program_id(0)
    k1 = jnp.maximum(
        jnp.dot(ea_ref[...], kW1_ref[...],
                preferred_element_type=jnp.float32) + kb1_ref[...], 0.0)
    k2 = jnp.maximum(
        jnp.dot(k1, kW2_ref[...],
                preferred_element_type=jnp.float32) + kb2_ref[...], 0.0)
    kmat = jnp.dot(k2.astype(jnp.bfloat16), kW3_ref[...],
                   preferred_element_type=jnp.float32) + kb3_ref[...]
    es = es_ref[...]
    hcol = es[:, :WIDTH]
    tiled = jnp.concatenate([hcol] * WIDTH, axis=1)
    msg = jnp.dot(kmat * tiled, D_ref[...],
                  preferred_element_type=jnp.float32)
    cf = jnp.maximum(
        jnp.dot(msg, cW1_ref[...],
                preferred_element_type=jnp.float32) + cb1_ref[...], 0.0)
    cfeat = jnp.dot(cf, cW2_ref[...],
                    preferred_element_type=jnp.float32) + cb2_ref[...]
    ccol = es[:, WIDTH:WIDTH + CDIM]
    eidx = i * BE + lax.broadcasted_iota(jnp.int32, (BE, 1), 0)
    valid = (eidx < E).astype(jnp.float32)
    payload = jnp.concatenate(
        [msg, cfeat, ccol * cfeat,
         jnp.ones((BE, 1), jnp.float32),
         jnp.zeros((BE, SW - WIDTH - CDIM - 2), jnp.float32)], axis=1)
    out_ref[...] = payload * valid


def _node_update(state, acc, nW, nb):
    a = (acc[0] + acc[1])[:N]
    h = state[:, :WIDTH]
    coord = state[:, WIDTH:WIDTH + CDIM]
    inv = 1.0 / jnp.maximum(a[:, 36:37], 1.0)
    h2 = jnp.maximum(
        jnp.dot(h, nW, preferred_element_type=jnp.float32) + nb
        + a[:, :WIDTH] * inv, 0.0)
    coord2 = coord + (coord * a[:, 32:33] - a[:, 33:36]) * inv
    return h2, coord2


def _update_body(state_ref, acc_ref, nW_ref, nb_ref, out_ref):
    h2, coord2 = _node_update(state_ref[...], acc_ref[...], nW_ref[...],
                              nb_ref[...])
    pad = jnp.zeros((N, SW - WIDTH - CDIM), jnp.float32)
    out_ref[...] = jnp.concatenate([h2, coord2, pad], axis=1)


def _final_body(state_ref, acc_ref, nW_ref, nb_ref, fW1_ref, fb1_ref,
                fW2_ref, fb2_ref, out_ref, coord_ref):
    h2, coord2 = _node_update(state_ref[...], acc_ref[...], nW_ref[...],
                              nb_ref[...])
    t = jnp.maximum(
        jnp.dot(h2, fW1_ref[...],
                preferred_element_type=jnp.float32) + fb1_ref[...], 0.0)
    out_ref[...] = jnp.dot(t, fW2_ref[...],
                           preferred_element_type=jnp.float32) + fb2_ref[...]
    coord_ref[...] = coord2


def _tc_init(x, coords, W1, b1):
    return pl.pallas_call(
        _init_body,
        out_shape=jax.ShapeDtypeStruct((N, SW), jnp.float32),
        grid=(1,),
        in_specs=[_full((N, 6)), _full((N, CDIM)), _full((6, WIDTH)),
                  _full((1, WIDTH))],
        out_specs=_full((N, SW)),
    )(x, coords, W1, b1)


def _tc_payload(ea, es, kW1, kb1, kW2, kb2, kW3, kb3, cW1, cb1, cW2, cb2, D):
    return pl.pallas_call(
        _payload_body,
        out_shape=jax.ShapeDtypeStruct((EPAD, SW), jnp.float32),
        grid=(GBLKS,),
        in_specs=[
            pl.BlockSpec((BE, 4), lambda i: (i, 0)),
            pl.BlockSpec((BE, SW), lambda i: (i, 0)),
            _full((4, 64)), _full((1, 64)),
            _full((64, 128)), _full((1, 128)),
            _full((128, 1024)), _full((1, 1024)),
            _full((WIDTH, WIDTH)), _full((1, WIDTH)),
            _full((WIDTH, 1)), _full((1, 1)),
            _full((1024, WIDTH)),
        ],
        out_specs=pl.BlockSpec((BE, SW), lambda i: (i, 0)),
    )(ea, es, kW1, kb1, kW2, kb2, kW3, kb3, cW1, cb1, cW2, cb2, D)


def _tc_update(state, acc, nW, nb):
    return pl.pallas_call(
        _update_body,
        out_shape=jax.ShapeDtypeStruct((N, SW), jnp.float32),
        grid=(1,),
        in_specs=[_full((N, SW)), _full((NC, NP, SW)),
                  _full((WIDTH, WIDTH)), _full((1, WIDTH))],
        out_specs=_full((N, SW)),
    )(state, acc, nW, nb)


def _tc_final(state, acc, nW, nb, fW1, fb1, fW2, fb2):
    return pl.pallas_call(
        _final_body,
        out_shape=(jax.ShapeDtypeStruct((N, 1), jnp.float32),
                   jax.ShapeDtypeStruct((N, CDIM), jnp.float32)),
        grid=(1,),
        in_specs=[_full((N, SW)), _full((NC, NP, SW)),
                  _full((WIDTH, WIDTH)), _full((1, WIDTH)),
                  _full((WIDTH, 2 * WIDTH)), _full((1, 2 * WIDTH)),
                  _full((2 * WIDTH, 1)), _full((1, 1))],
        out_specs=(_full((N, 1)), _full((N, CDIM))),
    )(state, acc, nW, nb, fW1, fb1, fW2, fb2)


@jax.jit
def kernel(x, edge_index, edge_attr, coords_init,
           W1, b1, kW1, kb1, kW2, kb2, kW3, kb3,
           nW, nb, cW1, cb1, cW2, cb2, fW1, fb1, fW2, fb2):
    row = edge_index[0]
    col = edge_index[1]
    col2d = jnp.pad(col, (0, EPAD - E)).reshape(EPAD // CHUNK, CHUNK)
    row2d = jnp.pad(row, (0, EPAD - E)).reshape(EPAD // CHUNK, CHUNK)
    ea = jnp.pad(edge_attr, ((0, EPAD - E), (0, 0)))
    zeros_stripe = jnp.zeros((NSTRIPE, SW), jnp.float32)
    D = jnp.repeat(jnp.eye(WIDTH, dtype=jnp.float32), WIDTH, axis=0)
    kW3b = kW3.astype(jnp.bfloat16)
    b1r = b1.reshape(1, -1)
    kb1r = kb1.reshape(1, -1)
    kb2r = kb2.reshape(1, -1)
    kb3r = kb3.reshape(1, -1)
    cb1r = cb1.reshape(1, -1)
    cb2r = cb2.reshape(1, -1)
    nbr = nb.reshape(1, -1)
    fb1r = fb1.reshape(1, -1)
    fb2r = fb2.reshape(1, -1)

    state = _tc_init(x, coords_init, W1, b1r)
    for d in range(2):
        es = _sc_gather(state, col2d)
        payload = _tc_payload(ea, es, kW1, kb1r, kW2, kb2r, kW3b, kb3r,
                              cW1, cb1r, cW2, cb2r, D)
        acc = _sc_scatter(payload, row2d, zeros_stripe)
        if d == 0:
            state = _tc_update(state, acc, nW, nbr)
        else:
            out, coord = _tc_final(state, acc, nW, nbr, fW1, fb1r,
                                   fW2, fb2r)
    return out, coord


# speedup vs baseline: 3.7313x; 1.0887x over previous
import functools

import jax
import jax.numpy as jnp
from jax import lax
from jax.experimental import pallas as pl
from jax.experimental.pallas import tpu as pltpu
from jax.experimental.pallas import tpu_sc as plsc

N = 10000
E = 160000
WIDTH = 32
SW = 48
CDIM = 3

NC = 2
NS = 16
NW = NC * NS
EPAD = 163840
EPW = EPAD // NW
CHUNK = 128
NCHUNK = EPW // CHUNK
KDEPTH = 8
NP = 10240
NSTRIPE = NP // NS

BE = 4096
GBLKS = EPAD // BE

_MESH = plsc.VectorSubcoreMesh(core_axis_name="c", subcore_axis_name="s")


@functools.partial(
    pl.kernel,
    out_type=jax.ShapeDtypeStruct((EPAD, SW), jnp.float32),
    mesh=_MESH,
    compiler_params=pltpu.CompilerParams(use_tc_tiling_on_sc=False),
    scratch_types=[
        pltpu.VMEM((NCHUNK, CHUNK), jnp.int32),
        pltpu.VMEM((KDEPTH, CHUNK, SW), jnp.float32),
        pltpu.SemaphoreType.DMA,
        pltpu.SemaphoreType.DMA,
    ],
)
def _sc_gather(state_hbm, col2d_hbm, out_hbm, idx_v, buf_v, gsem, ssem):
    w = lax.axis_index("s") * NC + lax.axis_index("c")
    pltpu.sync_copy(col2d_hbm.at[pl.ds(w * NCHUNK, NCHUNK)], idx_v)
    ebase = w * EPW
    LAG = KDEPTH - 1
    gds = [None] * NCHUNK
    sds = [None] * NCHUNK

    def _store(i):
        return pltpu.async_copy(
            buf_v.at[i % KDEPTH],
            out_hbm.at[pl.ds(ebase + i * CHUNK, CHUNK)], ssem)

    for j in range(NCHUNK):
        if j >= KDEPTH:
            sds[j - KDEPTH].wait()
        gds[j] = pltpu.async_copy(state_hbm.at[idx_v.at[j]],
                                  buf_v.at[j % KDEPTH], gsem)
        i = j - LAG
        if i >= 0:
            gds[i].wait()
            sds[i] = _store(i)
    for i in range(NCHUNK - LAG, NCHUNK):
        gds[i].wait()
        sds[i] = _store(i)
    for i in range(NCHUNK - KDEPTH, NCHUNK):
        sds[i].wait()


@functools.partial(
    pl.kernel,
    out_type=jax.ShapeDtypeStruct((NC, NP, SW), jnp.float32),
    mesh=_MESH,
    compiler_params=pltpu.CompilerParams(use_tc_tiling_on_sc=False),
    scratch_types=[
        pltpu.VMEM((NCHUNK, CHUNK), jnp.int32),
        pltpu.VMEM((KDEPTH, CHUNK, SW), jnp.float32),
        pltpu.VMEM((NSTRIPE, SW), jnp.float32),
        pltpu.VMEM_SHARED((NP, SW), jnp.float32),
        pltpu.SemaphoreType.DMA,
        pltpu.SemaphoreType.DMA,
    ],
)
def _sc_scatter(payload_hbm, row2d_hbm, zeros_hbm, out_hbm,
                idx_v, buf_v, stripe_v, acc_sh, lsem, asem):
    c = lax.axis_index("c")
    s = lax.axis_index("s")
    w = s * NC + c
    pltpu.sync_copy(zeros_hbm, stripe_v)
    pltpu.sync_copy(stripe_v, acc_sh.at[pl.ds(s * NSTRIPE, NSTRIPE)])
    pltpu.sync_copy(row2d_hbm.at[pl.ds(w * NCHUNK, NCHUNK)], idx_v)
    plsc.subcore_barrier()
    ebase = w * EPW
    for g in range(NCHUNK // KDEPTH):
        lds = []
        for b in range(KDEPTH):
            j = g * KDEPTH + b
            lds.append(
                pltpu.async_copy(
                    payload_hbm.at[pl.ds(ebase + j * CHUNK, CHUNK)],
                    buf_v.at[b], lsem))
        for d in lds:
            d.wait()
        ads = []
        for b in range(KDEPTH):
            j = g * KDEPTH + b
            ads.append(
                pltpu.async_copy(buf_v.at[b], acc_sh.at[idx_v.at[j]], asem,
                                 add=True))
        for d in ads:
            d.wait()
    plsc.subcore_barrier()
    pltpu.sync_copy(acc_sh.at[pl.ds(s * NSTRIPE, NSTRIPE)], stripe_v)
    pltpu.sync_copy(stripe_v, out_hbm.at[c, pl.ds(s * NSTRIPE, NSTRIPE)])


def _full(shape):
    return pl.BlockSpec(shape, lambda i: tuple(0 for _ in shape))


def _init_body(x_ref, coords_ref, W1_ref, b1_ref, state_ref):
    h = jnp.dot(x_ref[...], W1_ref[...],
                preferred_element_type=jnp.float32) + b1_ref[...]
    pad = jnp.zeros((N, SW - WIDTH - CDIM), jnp.float32)
    state_ref[...] = jnp.concatenate([h, coords_ref[...], pad], axis=1)


def _payload_body(ea_ref, es_ref, kW1_ref, kb1_ref, kW2_ref, kb2_ref,
                  kW3_ref, kb3_ref, cW1_ref, cb1_ref, cW2_ref, cb2_ref,
                  D_ref, out_ref):
    i = pl.program_id(0)
    k1 = jnp.maximum(
        jnp.dot(ea_ref[...], kW1_ref[...],
                preferred_element_type=jnp.float32) + kb1_ref[...], 0.0)
    k2 = jnp.maximum(
        jnp.dot(k1, kW2_ref[...],
                preferred_element_type=jnp.float32) + kb2_ref[...], 0.0)
    kmat = jnp.dot(k2.astype(jnp.bfloat16), kW3_ref[...],
                   preferred_element_type=jnp.float32) + kb3_ref[...]
    es = es_ref[...]
    hcol = es[:, :WIDTH]
    tiled = jnp.concatenate([hcol] * WIDTH, axis=1)
    msg = jnp.dot(kmat * tiled, D_ref[...],
                  preferred_element_type=jnp.float32)
    cf = jnp.maximum(
        jnp.dot(msg, cW1_ref[...],
                preferred_element_type=jnp.float32) + cb1_ref[...], 0.0)
    cfeat = jnp.dot(cf, cW2_ref[...],
                    preferred_element_type=jnp.float32) + cb2_ref[...]
    ccol = es[:, WIDTH:WIDTH + CDIM]
    eidx = i * BE + lax.broadcasted_iota(jnp.int32, (BE, 1), 0)
    valid = (eidx < E).astype(jnp.float32)
    payload = jnp.concatenate(
        [msg, cfeat, ccol * cfeat,
         jnp.ones((BE, 1), jnp.float32),
         jnp.zeros((BE, SW - WIDTH - CDIM - 2), jnp.float32)], axis=1)
    out_ref[...] = payload * valid


def _node_update(state, acc, nW, nb):
    a = (acc[0] + acc[1])[:N]
    h = state[:, :WIDTH]
    coord = state[:, WIDTH:WIDTH + CDIM]
    inv = 1.0 / jnp.maximum(a[:, 36:37], 1.0)
    h2 = jnp.maximum(
        jnp.dot(h, nW, preferred_element_type=jnp.float32) + nb
        + a[:, :WIDTH] * inv, 0.0)
    coord2 = coord + (coord * a[:, 32:33] - a[:, 33:36]) * inv
    return h2, coord2


def _update_body(state_ref, acc_ref, nW_ref, nb_ref, out_ref):
    h2, coord2 = _node_update(state_ref[...], acc_ref[...], nW_ref[...],
                              nb_ref[...])
    pad = jnp.zeros((N, SW - WIDTH - CDIM), jnp.float32)
    out_ref[...] = jnp.concatenate([h2, coord2, pad], axis=1)


def _final_body(state_ref, acc_ref, nW_ref, nb_ref, fW1_ref, fb1_ref,
                fW2_ref, fb2_ref, out_ref, coord_ref):
    h2, coord2 = _node_update(state_ref[...], acc_ref[...], nW_ref[...],
                              nb_ref[...])
    t = jnp.maximum(
        jnp.dot(h2, fW1_ref[...],
                preferred_element_type=jnp.float32) + fb1_ref[...], 0.0)
    out_ref[...] = jnp.dot(t, fW2_ref[...],
                           preferred_element_type=jnp.float32) + fb2_ref[...]
    coord_ref[...] = coord2


def _tc_init(x, coords, W1, b1):
    return pl.pallas_call(
        _init_body,
        out_shape=jax.ShapeDtypeStruct((N, SW), jnp.float32),
        grid=(1,),
        in_specs=[_full((N, 6)), _full((N, CDIM)), _full((6, WIDTH)),
                  _full((1, WIDTH))],
        out_specs=_full((N, SW)),
    )(x, coords, W1, b1)


def _tc_payload(ea, es, kW1, kb1, kW2, kb2, kW3, kb3, cW1, cb1, cW2, cb2, D):
    return pl.pallas_call(
        _payload_body,
        out_shape=jax.ShapeDtypeStruct((EPAD, SW), jnp.float32),
        grid=(GBLKS,),
        in_specs=[
            pl.BlockSpec((BE, 4), lambda i: (i, 0)),
            pl.BlockSpec((BE, SW), lambda i: (i, 0)),
            _full((4, 64)), _full((1, 64)),
            _full((64, 128)), _full((1, 128)),
            _full((128, 1024)), _full((1, 1024)),
            _full((WIDTH, WIDTH)), _full((1, WIDTH)),
            _full((WIDTH, 1)), _full((1, 1)),
            _full((1024, WIDTH)),
        ],
        out_specs=pl.BlockSpec((BE, SW), lambda i: (i, 0)),
    )(ea, es, kW1, kb1, kW2, kb2, kW3, kb3, cW1, cb1, cW2, cb2, D)


def _tc_update(state, acc, nW, nb):
    return pl.pallas_call(
        _update_body,
        out_shape=jax.ShapeDtypeStruct((N, SW), jnp.float32),
        grid=(1,),
        in_specs=[_full((N, SW)), _full((NC, NP, SW)),
                  _full((WIDTH, WIDTH)), _full((1, WIDTH))],
        out_specs=_full((N, SW)),
    )(state, acc, nW, nb)


def _tc_final(state, acc, nW, nb, fW1, fb1, fW2, fb2):
    return pl.pallas_call(
        _final_body,
        out_shape=(jax.ShapeDtypeStruct((N, 1), jnp.float32),
                   jax.ShapeDtypeStruct((N, CDIM), jnp.float32)),
        grid=(1,),
        in_specs=[_full((N, SW)), _full((NC, NP, SW)),
                  _full((WIDTH, WIDTH)), _full((1, WIDTH)),
                  _full((WIDTH, 2 * WIDTH)), _full((1, 2 * WIDTH)),
                  _full((2 * WIDTH, 1)), _full((1, 1))],
        out_specs=(_full((N, 1)), _full((N, CDIM))),
    )(state, acc, nW, nb, fW1, fb1, fW2, fb2)


@jax.jit
def kernel(x, edge_index, edge_attr, coords_init,
           W1, b1, kW1, kb1, kW2, kb2, kW3, kb3,
           nW, nb, cW1, cb1, cW2, cb2, fW1, fb1, fW2, fb2):
    row = edge_index[0]
    col = edge_index[1]
    col2d = jnp.pad(col, (0, EPAD - E)).reshape(EPAD // CHUNK, CHUNK)
    row2d = jnp.pad(row, (0, EPAD - E)).reshape(EPAD // CHUNK, CHUNK)
    ea = jnp.pad(edge_attr, ((0, EPAD - E), (0, 0)))
    zeros_stripe = jnp.zeros((NSTRIPE, SW), jnp.float32)
    D = jnp.repeat(jnp.eye(WIDTH, dtype=jnp.float32), WIDTH, axis=0)
    kW3b = kW3.astype(jnp.bfloat16)
    kb3r = kb3.reshape(1, -1)
    b1r = b1.reshape(1, -1)
    kb1r = kb1.reshape(1, -1)
    kb2r = kb2.reshape(1, -1)
    cb1r = cb1.reshape(1, -1)
    cb2r = cb2.reshape(1, -1)
    nbr = nb.reshape(1, -1)
    fb1r = fb1.reshape(1, -1)
    fb2r = fb2.reshape(1, -1)

    state = _tc_init(x, coords_init, W1, b1r)
    for d in range(2):
        es = _sc_gather(state, col2d)
        payload = _tc_payload(ea, es, kW1, kb1r, kW2, kb2r, kW3b, kb3r,
                              cW1, cb1r, cW2, cb2r, D)
        acc = _sc_scatter(payload, row2d, zeros_stripe)
        if d == 0:
            state = _tc_update(state, acc, nW, nbr)
        else:
            out, coord = _tc_final(state, acc, nW, nbr, fW1, fb1r,
                                   fW2, fb2r)
    return out, coord


# speedup vs baseline: 3.7330x; 1.0005x over previous
import functools

import jax
import jax.numpy as jnp
from jax import lax
from jax.experimental import pallas as pl
from jax.experimental.pallas import tpu as pltpu
from jax.experimental.pallas import tpu_sc as plsc

N = 10000
E = 160000
WIDTH = 32
SW = 48
CDIM = 3

NC = 2
NS = 16
NW = NC * NS
EPAD = 163840
EPW = EPAD // NW
CHUNK = 128
NCHUNK = EPW // CHUNK
GK = 16
KDEPTH = 8
NP = 10240
NSTRIPE = NP // NS

BE = 4096
GBLKS = EPAD // BE

_MESH = plsc.VectorSubcoreMesh(core_axis_name="c", subcore_axis_name="s")


@functools.partial(
    pl.kernel,
    out_type=jax.ShapeDtypeStruct((EPAD, SW), jnp.float32),
    mesh=_MESH,
    compiler_params=pltpu.CompilerParams(use_tc_tiling_on_sc=False),
    scratch_types=[
        pltpu.VMEM((NCHUNK, CHUNK), jnp.int32),
        pltpu.VMEM((GK, CHUNK, SW), jnp.float32),
        pltpu.SemaphoreType.DMA,
        pltpu.SemaphoreType.DMA,
    ],
)
def _sc_gather(state_hbm, col2d_hbm, out_hbm, idx_v, buf_v, gsem, ssem):
    w = lax.axis_index("s") * NC + lax.axis_index("c")
    pltpu.sync_copy(col2d_hbm.at[pl.ds(w * NCHUNK, NCHUNK)], idx_v)
    ebase = w * EPW
    LAG = GK - 1
    gds = [None] * NCHUNK
    sds = [None] * NCHUNK

    def _store(i):
        return pltpu.async_copy(
            buf_v.at[i % GK],
            out_hbm.at[pl.ds(ebase + i * CHUNK, CHUNK)], ssem)

    for j in range(NCHUNK):
        if j >= GK:
            sds[j - GK].wait()
        gds[j] = pltpu.async_copy(state_hbm.at[idx_v.at[j]],
                                  buf_v.at[j % GK], gsem)
        i = j - LAG
        if i >= 0:
            gds[i].wait()
            sds[i] = _store(i)
    for i in range(NCHUNK - LAG, NCHUNK):
        gds[i].wait()
        sds[i] = _store(i)
    for i in range(NCHUNK - GK, NCHUNK):
        sds[i].wait()


@functools.partial(
    pl.kernel,
    out_type=jax.ShapeDtypeStruct((NC, NP, SW), jnp.float32),
    mesh=_MESH,
    compiler_params=pltpu.CompilerParams(use_tc_tiling_on_sc=False),
    scratch_types=[
        pltpu.VMEM((NCHUNK, CHUNK), jnp.int32),
        pltpu.VMEM((KDEPTH, CHUNK, SW), jnp.float32),
        pltpu.VMEM((NSTRIPE, SW), jnp.float32),
        pltpu.VMEM_SHARED((NP, SW), jnp.float32),
        pltpu.SemaphoreType.DMA,
        pltpu.SemaphoreType.DMA,
    ],
)
def _sc_scatter(payload_hbm, row2d_hbm, zeros_hbm, out_hbm,
                idx_v, buf_v, stripe_v, acc_sh, lsem, asem):
    c = lax.axis_index("c")
    s = lax.axis_index("s")
    w = s * NC + c
    pltpu.sync_copy(zeros_hbm, stripe_v)
    pltpu.sync_copy(stripe_v, acc_sh.at[pl.ds(s * NSTRIPE, NSTRIPE)])
    pltpu.sync_copy(row2d_hbm.at[pl.ds(w * NCHUNK, NCHUNK)], idx_v)
    plsc.subcore_barrier()
    ebase = w * EPW
    for g in range(NCHUNK // KDEPTH):
        lds = []
        for b in range(KDEPTH):
            j = g * KDEPTH + b
            lds.append(
                pltpu.async_copy(
                    payload_hbm.at[pl.ds(ebase + j * CHUNK, CHUNK)],
                    buf_v.at[b], lsem))
        for d in lds:
            d.wait()
        ads = []
        for b in range(KDEPTH):
            j = g * KDEPTH + b
            ads.append(
                pltpu.async_copy(buf_v.at[b], acc_sh.at[idx_v.at[j]], asem,
                                 add=True))
        for d in ads:
            d.wait()
    plsc.subcore_barrier()
    pltpu.sync_copy(acc_sh.at[pl.ds(s * NSTRIPE, NSTRIPE)], stripe_v)
    pltpu.sync_copy(stripe_v, out_hbm.at[c, pl.ds(s * NSTRIPE, NSTRIPE)])


def _full(shape):
    return pl.BlockSpec(shape, lambda i: tuple(0 for _ in shape))


def _init_body(x_ref, coords_ref, W1_ref, b1_ref, state_ref):
    h = jnp.dot(x_ref[...], W1_ref[...],
                preferred_element_type=jnp.float32) + b1_ref[...]
    pad = jnp.zeros((N, SW - WIDTH - CDIM), jnp.float32)
    state_ref[...] = jnp.concatenate([h, coords_ref[...], pad], axis=1)


def _payload_body(ea_ref, es_ref, kW1_ref, kb1_ref, kW2_ref, kb2_ref,
                  kW3_ref, kb3_ref, cW1_ref, cb1_ref, cW2_ref, cb2_ref,
                  D_ref, out_ref):
    i = pl.program_id(0)
    k1 = jnp.maximum(
        jnp.dot(ea_ref[...], kW1_ref[...],
                preferred_element_type=jnp.float32) + kb1_ref[...], 0.0)
    k2 = jnp.maximum(
        jnp.dot(k1, kW2_ref[...],
                preferred_element_type=jnp.float32) + kb2_ref[...], 0.0)
    kmat = jnp.dot(k2.astype(jnp.bfloat16), kW3_ref[...],
                   preferred_element_type=jnp.float32) + kb3_ref[...]
    es = es_ref[...]
    hcol = es[:, :WIDTH]
    tiled = jnp.concatenate([hcol] * WIDTH, axis=1)
    msg = jnp.dot(kmat * tiled, D_ref[...],
                  preferred_element_type=jnp.float32)
    cf = jnp.maximum(
        jnp.dot(msg, cW1_ref[...],
                preferred_element_type=jnp.float32) + cb1_ref[...], 0.0)
    cfeat = jnp.dot(cf, cW2_ref[...],
                    preferred_element_type=jnp.float32) + cb2_ref[...]
    ccol = es[:, WIDTH:WIDTH + CDIM]
    eidx = i * BE + lax.broadcasted_iota(jnp.int32, (BE, 1), 0)
    valid = (eidx < E).astype(jnp.float32)
    payload = jnp.concatenate(
        [msg, cfeat, ccol * cfeat,
         jnp.ones((BE, 1), jnp.float32),
         jnp.zeros((BE, SW - WIDTH - CDIM - 2), jnp.float32)], axis=1)
    out_ref[...] = payload * valid


def _node_update(state, acc, nW, nb):
    a = (acc[0] + acc[1])[:N]
    h = state[:, :WIDTH]
    coord = state[:, WIDTH:WIDTH + CDIM]
    inv = 1.0 / jnp.maximum(a[:, 36:37], 1.0)
    h2 = jnp.maximum(
        jnp.dot(h, nW, preferred_element_type=jnp.float32) + nb
        + a[:, :WIDTH] * inv, 0.0)
    coord2 = coord + (coord * a[:, 32:33] - a[:, 33:36]) * inv
    return h2, coord2


def _update_body(state_ref, acc_ref, nW_ref, nb_ref, out_ref):
    h2, coord2 = _node_update(state_ref[...], acc_ref[...], nW_ref[...],
                              nb_ref[...])
    pad = jnp.zeros((N, SW - WIDTH - CDIM), jnp.float32)
    out_ref[...] = jnp.concatenate([h2, coord2, pad], axis=1)


def _final_body(state_ref, acc_ref, nW_ref, nb_ref, fW1_ref, fb1_ref,
                fW2_ref, fb2_ref, out_ref, coord_ref):
    h2, coord2 = _node_update(state_ref[...], acc_ref[...], nW_ref[...],
                              nb_ref[...])
    t = jnp.maximum(
        jnp.dot(h2, fW1_ref[...],
                preferred_element_type=jnp.float32) + fb1_ref[...], 0.0)
    out_ref[...] = jnp.dot(t, fW2_ref[...],
                           preferred_element_type=jnp.float32) + fb2_ref[...]
    coord_ref[...] = coord2


def _tc_init(x, coords, W1, b1):
    return pl.pallas_call(
        _init_body,
        out_shape=jax.ShapeDtypeStruct((N, SW), jnp.float32),
        grid=(1,),
        in_specs=[_full((N, 6)), _full((N, CDIM)), _full((6, WIDTH)),
                  _full((1, WIDTH))],
        out_specs=_full((N, SW)),
    )(x, coords, W1, b1)


def _tc_payload(ea, es, kW1, kb1, kW2, kb2, kW3, kb3, cW1, cb1, cW2, cb2, D):
    return pl.pallas_call(
        _payload_body,
        out_shape=jax.ShapeDtypeStruct((EPAD, SW), jnp.float32),
        grid=(GBLKS,),
        in_specs=[
            pl.BlockSpec((BE, 4), lambda i: (i, 0)),
            pl.BlockSpec((BE, SW), lambda i: (i, 0)),
            _full((4, 64)), _full((1, 64)),
            _full((64, 128)), _full((1, 128)),
            _full((128, 1024)), _full((1, 1024)),
            _full((WIDTH, WIDTH)), _full((1, WIDTH)),
            _full((WIDTH, 1)), _full((1, 1)),
            _full((1024, WIDTH)),
        ],
        out_specs=pl.BlockSpec((BE, SW), lambda i: (i, 0)),
    )(ea, es, kW1, kb1, kW2, kb2, kW3, kb3, cW1, cb1, cW2, cb2, D)


def _tc_update(state, acc, nW, nb):
    return pl.pallas_call(
        _update_body,
        out_shape=jax.ShapeDtypeStruct((N, SW), jnp.float32),
        grid=(1,),
        in_specs=[_full((N, SW)), _full((NC, NP, SW)),
                  _full((WIDTH, WIDTH)), _full((1, WIDTH))],
        out_specs=_full((N, SW)),
    )(state, acc, nW, nb)


def _tc_final(state, acc, nW, nb, fW1, fb1, fW2, fb2):
    return pl.pallas_call(
        _final_body,
        out_shape=(jax.ShapeDtypeStruct((N, 1), jnp.float32),
                   jax.ShapeDtypeStruct((N, CDIM), jnp.float32)),
        grid=(1,),
        in_specs=[_full((N, SW)), _full((NC, NP, SW)),
                  _full((WIDTH, WIDTH)), _full((1, WIDTH)),
                  _full((WIDTH, 2 * WIDTH)), _full((1, 2 * WIDTH)),
                  _full((2 * WIDTH, 1)), _full((1, 1))],
        out_specs=(_full((N, 1)), _full((N, CDIM))),
    )(state, acc, nW, nb, fW1, fb1, fW2, fb2)


@jax.jit
def kernel(x, edge_index, edge_attr, coords_init,
           W1, b1, kW1, kb1, kW2, kb2, kW3, kb3,
           nW, nb, cW1, cb1, cW2, cb2, fW1, fb1, fW2, fb2):
    row = edge_index[0]
    col = edge_index[1]
    col2d = jnp.pad(col, (0, EPAD - E)).reshape(EPAD // CHUNK, CHUNK)
    row2d = jnp.pad(row, (0, EPAD - E)).reshape(EPAD // CHUNK, CHUNK)
    ea = jnp.pad(edge_attr, ((0, EPAD - E), (0, 0)))
    zeros_stripe = jnp.zeros((NSTRIPE, SW), jnp.float32)
    D = jnp.repeat(jnp.eye(WIDTH, dtype=jnp.float32), WIDTH, axis=0)
    kW3b = kW3.astype(jnp.bfloat16)
    kb3r = kb3.reshape(1, -1)
    b1r = b1.reshape(1, -1)
    kb1r = kb1.reshape(1, -1)
    kb2r = kb2.reshape(1, -1)
    cb1r = cb1.reshape(1, -1)
    cb2r = cb2.reshape(1, -1)
    nbr = nb.reshape(1, -1)
    fb1r = fb1.reshape(1, -1)
    fb2r = fb2.reshape(1, -1)

    state = _tc_init(x, coords_init, W1, b1r)
    for d in range(2):
        es = _sc_gather(state, col2d)
        payload = _tc_payload(ea, es, kW1, kb1r, kW2, kb2r, kW3b, kb3r,
                              cW1, cb1r, cW2, cb2r, D)
        acc = _sc_scatter(payload, row2d, zeros_stripe)
        if d == 0:
            state = _tc_update(state, acc, nW, nbr)
        else:
            out, coord = _tc_final(state, acc, nW, nbr, fW1, fb1r,
                                   fW2, fb2r)
    return out, coord
